# baseline (speedup 1.0000x reference)
import functools

import jax
import jax.numpy as jnp
from jax import lax
from jax.experimental import pallas as pl
from jax.experimental.pallas import tpu as pltpu
from jax.experimental.pallas import tpu_sc as plsc

NC = 2
NS = 16
NW = NC * NS
ENT_D = 64
REL_ROW = 272


def _sc_gather(ent_tab, rel_tab, ent_idx, rel_idx):
    _, je, bpw = ent_idx.shape
    _, jr, _ = rel_idx.shape

    mesh = plsc.VectorSubcoreMesh(core_axis_name="c", subcore_axis_name="s")

    @functools.partial(
        pl.kernel,
        mesh=mesh,
        compiler_params=pltpu.CompilerParams(use_tc_tiling_on_sc=False),
        out_type=[
            jax.ShapeDtypeStruct((NW, je, bpw, ENT_D), jnp.float32),
            jax.ShapeDtypeStruct((NW, jr, bpw, REL_ROW), jnp.float32),
        ],
        scratch_types=[
            pltpu.VMEM((je, bpw), jnp.int32),
            pltpu.VMEM((jr, bpw), jnp.int32),
            pltpu.VMEM((je, bpw, ENT_D), jnp.float32),
            pltpu.VMEM((jr, bpw, REL_ROW), jnp.float32),
            pltpu.SemaphoreType.DMA,
        ],
    )
    def k(ent_hbm, rel_hbm, eidx_hbm, ridx_hbm, eout_hbm, rout_hbm,
          eidx_v, ridx_v, erows_v, rrows_v, sem):
        wid = lax.axis_index("s") * NC + lax.axis_index("c")
        pltpu.sync_copy(eidx_hbm.at[wid], eidx_v)
        pltpu.sync_copy(ridx_hbm.at[wid], ridx_v)
        copies = []
        for j in range(je):
            copies.append(
                pltpu.async_copy(ent_hbm.at[eidx_v.at[j]], erows_v.at[j], sem))
        for j in range(jr):
            copies.append(
                pltpu.async_copy(rel_hbm.at[ridx_v.at[j]], rrows_v.at[j], sem))
        for c in copies:
            c.wait()
        pltpu.sync_copy(erows_v, eout_hbm.at[wid])
        pltpu.sync_copy(rrows_v, rout_hbm.at[wid])

    return k(ent_tab, rel_tab, ent_idx, rel_idx)


def _tc_body(ph, pt, nh, nt, pr, nr, out):
    def l2n(x):
        ss = jnp.sum(x * x, axis=-1, keepdims=True)
        return x * lax.rsqrt(jnp.maximum(ss, 1e-12))

    def neg_log_score(h_raw, t_raw, rw):
        h = l2n(h_raw[...])
        t = l2n(t_raw[...])
        r = rw[...]
        w = r[:, 4 * ENT_D:4 * ENT_D + 4]
        wn = w * lax.rsqrt(
            jnp.maximum(jnp.sum(w * w, axis=-1, keepdims=True), 1e-12))
        ssum = None
        for c in range(4):
            rc = l2n(r[:, c * ENT_D:(c + 1) * ENT_D])
            d = rc + h - t
            n2 = jnp.sum(d * d, axis=-1, keepdims=True)
            term = wn[:, c:c + 1] * jnp.exp(n2)
            ssum = term if ssum is None else ssum + term
        return -jnp.log(jnp.maximum(ssum, 1e-8))

    p = neg_log_score(ph, pt, pr)
    n = neg_log_score(nh, nt, nr)
    blk = jnp.sum(jnp.maximum(p - n + 1.0, 0.0))

    @pl.when(pl.program_id(0) == 0)
    def _():
        out[...] = jnp.zeros((1, 1), jnp.float32)

    out[...] = out[...] + blk


def _tc_loss(ph_e, pt_e, nh_e, nt_e, pr_e, nr_e, blk):
    b = ph_e.shape[0]
    grid = (b // blk,)
    ent_spec = pl.BlockSpec((blk, ENT_D), lambda i: (i, 0))
    rel_spec = pl.BlockSpec((blk, REL_ROW), lambda i: (i, 0))
    return pl.pallas_call(
        _tc_body,
        grid=grid,
        in_specs=[ent_spec, ent_spec, ent_spec, ent_spec, rel_spec, rel_spec],
        out_specs=pl.BlockSpec((1, 1), lambda i: (0, 0)),
        out_shape=jax.ShapeDtypeStruct((1, 1), jnp.float32),
    )(ph_e, pt_e, nh_e, nt_e, pr_e, nr_e)


def kernel(pos_h, pos_t, pos_r, neg_h, neg_t, neg_r,
           ent_embeddings, rel_embeddings, rel_weights):
    b = pos_h.shape[0]
    rel_total, clus, rel_d = rel_embeddings.shape

    rel_tab = jnp.concatenate(
        [rel_embeddings.reshape(rel_total, clus * rel_d),
         rel_weights,
         jnp.zeros((rel_total, REL_ROW - clus * rel_d - clus), jnp.float32)],
        axis=1)

    ent_idx = jnp.concatenate(
        [pos_h, pos_t, neg_h, neg_t], axis=0).astype(jnp.int32)
    rel_idx = jnp.concatenate([pos_r, neg_r], axis=0).astype(jnp.int32)
    bpw = b // NW
    ent_idx = ent_idx.reshape(NW, 4, bpw)
    rel_idx = rel_idx.reshape(NW, 2, bpw)

    ent_rows, rel_rows = _sc_gather(ent_embeddings, rel_tab, ent_idx, rel_idx)
    ent_rows = ent_rows.reshape(4 * b, ENT_D)
    rel_rows = rel_rows.reshape(2 * b, REL_ROW)

    loss = _tc_loss(ent_rows[0:b], ent_rows[b:2 * b],
                    ent_rows[2 * b:3 * b], ent_rows[3 * b:4 * b],
                    rel_rows[0:b], rel_rows[b:2 * b], blk=2048)
    return loss[0, 0]


# speedup vs baseline: 1.6536x; 1.6536x over previous
import functools

import jax
import jax.numpy as jnp
from jax import lax
from jax.experimental import pallas as pl
from jax.experimental.pallas import tpu as pltpu
from jax.experimental.pallas import tpu_sc as plsc

NC = 2
NS = 16
NW = NC * NS
ENT_D = 64
REL_ROW = 384
ECHUNK = 128


def _sc_gather(ent_tab, rel_tab, ent_idx, rel_idx):
    _, epw = ent_idx.shape
    _, jr, rpc = rel_idx.shape

    mesh = plsc.VectorSubcoreMesh(core_axis_name="c", subcore_axis_name="s")

    @functools.partial(
        pl.kernel,
        mesh=mesh,
        compiler_params=pltpu.CompilerParams(use_tc_tiling_on_sc=True),
        out_type=[
            jax.ShapeDtypeStruct((NW * epw, ENT_D), jnp.float32),
            jax.ShapeDtypeStruct((NW * jr * rpc, REL_ROW), jnp.float32),
        ],
        scratch_types=[
            pltpu.VMEM((epw,), jnp.int32),
            pltpu.VMEM((jr, rpc), jnp.int32),
            pltpu.VMEM((epw, ENT_D), jnp.float32),
            pltpu.VMEM((rpc, REL_ROW), jnp.float32),
            pltpu.SemaphoreType.DMA,
            pltpu.SemaphoreType.DMA,
        ],
    )
    def k(ent_hbm, rel_hbm, eidx_hbm, ridx_hbm, eout_hbm, rout_hbm,
          eidx_v, ridx_v, erows_v, rrows_v, esem, rsem):
        wid = lax.axis_index("s") * NC + lax.axis_index("c")
        pltpu.sync_copy(eidx_hbm.at[wid], eidx_v)
        pltpu.sync_copy(ridx_hbm.at[wid], ridx_v)

        def fire(g, carry):
            base = g * 16
            idx16 = eidx_v[pl.ds(base, 16)]
            for j in range(16):
                row = jnp.squeeze(lax.slice(idx16, (j,), (j + 1,)))
                pltpu.async_copy(ent_hbm.at[pl.ds(row, 1)],
                                 erows_v.at[pl.ds(base + j, 1)], esem)
            return carry

        for cstart in range(0, epw, ECHUNK):
            lax.fori_loop(cstart // 16, (cstart + ECHUNK) // 16, fire, 0)
            pltpu.make_async_copy(
                eout_hbm.at[pl.ds(0, ECHUNK)],
                erows_v.at[pl.ds(cstart, ECHUNK)], esem).wait()

        pltpu.sync_copy(erows_v, eout_hbm.at[pl.ds(wid * epw, epw)])

        for j in range(jr):
            pltpu.async_copy(rel_hbm.at[ridx_v.at[j]], rrows_v, rsem).wait()
            pltpu.sync_copy(
                rrows_v, rout_hbm.at[pl.ds((wid * jr + j) * rpc, rpc)])

    return k(ent_tab, rel_tab, ent_idx, rel_idx)


def _tc_body(ph, pt, nh, nt, pr, nr, out):
    def l2n(x):
        ss = jnp.sum(x * x, axis=-1, keepdims=True)
        return x * lax.rsqrt(jnp.maximum(ss, 1e-12))

    def neg_log_score(h_raw, t_raw, rw):
        h = l2n(h_raw[...])
        t = l2n(t_raw[...])
        r = rw[...]
        w = r[:, 4 * ENT_D:4 * ENT_D + 4]
        wn = w * lax.rsqrt(
            jnp.maximum(jnp.sum(w * w, axis=-1, keepdims=True), 1e-12))
        ssum = None
        for c in range(4):
            rc = l2n(r[:, c * ENT_D:(c + 1) * ENT_D])
            d = rc + h - t
            n2 = jnp.sum(d * d, axis=-1, keepdims=True)
            term = wn[:, c:c + 1] * jnp.exp(n2)
            ssum = term if ssum is None else ssum + term
        return -jnp.log(jnp.maximum(ssum, 1e-8))

    p = neg_log_score(ph, pt, pr)
    n = neg_log_score(nh, nt, nr)
    blk = jnp.sum(jnp.maximum(p - n + 1.0, 0.0))

    @pl.when(pl.program_id(0) == 0)
    def _():
        out[...] = jnp.zeros((1, 1), jnp.float32)

    out[...] = out[...] + blk


def _tc_loss(ph_e, pt_e, nh_e, nt_e, pr_e, nr_e, blk):
    b = ph_e.shape[0]
    grid = (b // blk,)
    ent_spec = pl.BlockSpec((blk, ENT_D), lambda i: (i, 0))
    rel_spec = pl.BlockSpec((blk, REL_ROW), lambda i: (i, 0))
    return pl.pallas_call(
        _tc_body,
        grid=grid,
        in_specs=[ent_spec, ent_spec, ent_spec, ent_spec, rel_spec, rel_spec],
        out_specs=pl.BlockSpec((1, 1), lambda i: (0, 0)),
        out_shape=jax.ShapeDtypeStruct((1, 1), jnp.float32),
    )(ph_e, pt_e, nh_e, nt_e, pr_e, nr_e)


def kernel(pos_h, pos_t, pos_r, neg_h, neg_t, neg_r,
           ent_embeddings, rel_embeddings, rel_weights):
    b = pos_h.shape[0]
    rel_total, clus, rel_d = rel_embeddings.shape

    rel_tab = jnp.concatenate(
        [rel_embeddings.reshape(rel_total, clus * rel_d),
         rel_weights,
         jnp.zeros((rel_total, REL_ROW - clus * rel_d - clus), jnp.float32)],
        axis=1)

    ent_idx = jnp.concatenate(
        [pos_h, pos_t, neg_h, neg_t], axis=0).astype(jnp.int32)
    rel_idx = jnp.concatenate([pos_r, neg_r], axis=0).astype(jnp.int32)
    epw = 4 * b // NW
    ent_idx = ent_idx.reshape(NW, epw)
    rel_idx = rel_idx.reshape(NW, (2 * b // NW) // 128, 128)

    ent_rows, rel_rows = _sc_gather(ent_embeddings, rel_tab, ent_idx, rel_idx)

    loss = _tc_loss(ent_rows[0:b], ent_rows[b:2 * b],
                    ent_rows[2 * b:3 * b], ent_rows[3 * b:4 * b],
                    rel_rows[0:b], rel_rows[b:2 * b], blk=2048)
    return loss[0, 0]
